# baseline (speedup 1.0000x reference)
import functools

import jax
import jax.numpy as jnp
from jax import lax
from jax.experimental import pallas as pl
from jax.experimental.pallas import tpu as pltpu
from jax.experimental.pallas import tpu_sc as plsc

NUM_USERS = 10000
EMBED_DIM = 128
NUM_EDGES = 320000
BATCH = 16384

NC = 2
NS = 16
NW = NC * NS

DCOL = EMBED_DIM + 16

ECH = 128
NCH = 80
EPW = NCH * ECH
NROW = NW * NCH
EPAD = NROW * ECH

UPW = BATCH // NW
UCH = 64
NUCH = UPW // UCH

UPAD = 10240
ROWS_PER_TILE = UPAD // NS
ZR = 128
SROW = NUM_USERS // NS
SCH = 125

EGRID = 32
EBLK = NROW // EGRID * ECH


def _edge_fmt_body(ei_ref, src_ref, dst_ref):
  i = pl.program_id(0)
  rows = EBLK // ECH
  s = ei_ref[0:1, :].reshape(rows, ECH)
  d = ei_ref[1:2, :].reshape(rows, ECH)
  flat = (i * EBLK
          + jax.lax.broadcasted_iota(jnp.int32, (rows, ECH), 0) * ECH
          + jax.lax.broadcasted_iota(jnp.int32, (rows, ECH), 1))
  valid = flat < NUM_EDGES
  src_ref[...] = jnp.where(valid, s, (flat * 37) % NUM_USERS)
  dst_ref[...] = jnp.where(valid, d,
                           NUM_USERS + flat % (UPAD - NUM_USERS))


def _edge_fmt(edge_index):
  i32 = jnp.int32
  return pl.pallas_call(
      _edge_fmt_body,
      grid=(EGRID,),
      in_specs=[pl.BlockSpec((2, EBLK), lambda i: (0, i))],
      out_specs=[pl.BlockSpec((EBLK // ECH, ECH), lambda i: (i, 0)),
                 pl.BlockSpec((EBLK // ECH, ECH), lambda i: (i, 0))],
      out_shape=[jax.ShapeDtypeStruct((NROW, ECH), i32),
                 jax.ShapeDtypeStruct((NROW, ECH), i32)],
  )(edge_index)


def _zero_fill(ref, rows, cols):
  z = jnp.zeros((16,), jnp.float32)

  def body(i, _):
    r = i // (cols // 16)
    cidx = (i % (cols // 16)) * 16
    ref[r, pl.ds(cidx, 16)] = z
    return 0

  lax.fori_loop(0, rows * (cols // 16), body, 0)


def _stage_a(src2, dst2, u2e):
  mesh = plsc.VectorSubcoreMesh(core_axis_name="c", subcore_axis_name="s")
  f32 = jnp.float32

  @functools.partial(
      pl.kernel,
      mesh=mesh,
      compiler_params=pltpu.CompilerParams(use_tc_tiling_on_sc=False),
      out_type=[
          jax.ShapeDtypeStruct((UPAD, DCOL), f32),
          jax.ShapeDtypeStruct((UPAD, DCOL), f32),
      ],
      scratch_types=[
          pltpu.HBM((NC, NUM_USERS, DCOL), f32),
          pltpu.VMEM_SHARED((UPAD, DCOL), f32),
          pltpu.VMEM((8, ECH), jnp.int32),
          pltpu.VMEM((8, ECH), jnp.int32),
          pltpu.VMEM((ECH, DCOL), f32),
          pltpu.VMEM((ECH, DCOL), f32),
          pltpu.SemaphoreType.DMA,
          pltpu.SemaphoreType.DMA,
          pltpu.SemaphoreType.DMA,
          pltpu.SemaphoreType.DMA,
          pltpu.SemaphoreType.DMA,
          pltpu.SemaphoreType.DMA,
          pltpu.SemaphoreType.DMA,
          pltpu.SemaphoreType.DMA,
      ],
  )
  def k(src_hbm, dst_hbm, u2e_hbm, aggc0_hbm, aggc1_hbm,
        u2ex_hbm, agg_sh, sbufs, dbufs, rows0, rows1,
        gsem0, gsem1, ssem0, ssem1, isem0, isem1, isem2, isem3):
    c = lax.axis_index("c")
    s = lax.axis_index("s")
    wid = s * NC + c
    roff = wid * NCH

    one = jnp.ones((16,), f32)

    def ones_body(i, _):
      rows1[i, pl.ds(EMBED_DIM, 16)] = one
      return 0

    lax.fori_loop(0, SCH, ones_body, 0)
    for j in range(SROW // SCH):
      srow = s * SROW + j * SCH
      pltpu.sync_copy(u2e_hbm.at[pl.ds(srow, SCH)],
                      rows1.at[pl.ds(0, SCH), pl.ds(0, EMBED_DIM)])
      pltpu.sync_copy(rows1.at[pl.ds(0, SCH)],
                      u2ex_hbm.at[c, pl.ds(srow, SCH)])

    _zero_fill(rows0, ZR, DCOL)
    base_row = s * ROWS_PER_TILE
    for j in range(ROWS_PER_TILE // ZR):
      pltpu.sync_copy(rows0, agg_sh.at[pl.ds(base_row + j * ZR, ZR)])
    plsc.subcore_barrier()

    tab = u2ex_hbm.at[c]
    isems = (isem0, isem1, isem2, isem3)
    gsems = (gsem0, gsem1)
    ssems = (ssem0, ssem1)
    rbufs = (rows0, rows1)

    def load_idx_sync(kk, b):
      pltpu.sync_copy(src_hbm.at[roff + kk], sbufs.at[b])
      pltpu.sync_copy(dst_hbm.at[roff + kk], dbufs.at[b])

    def load_idx_async(kk, b):
      pltpu.async_copy(src_hbm.at[roff + kk], sbufs.at[b], isems[b % 4])
      pltpu.async_copy(dst_hbm.at[roff + kk], dbufs.at[b], isems[b % 4])

    def wait_idx(b):
      pltpu.make_async_copy(src_hbm.at[roff], sbufs.at[b],
                            isems[b % 4]).wait()
      pltpu.make_async_copy(dst_hbm.at[roff], dbufs.at[b],
                            isems[b % 4]).wait()

    def gather(b, buf, sem):
      pltpu.async_copy(tab.at[sbufs.at[b]], buf, sem)

    def wait_gather(buf, sem):
      pltpu.make_async_copy(tab.at[sbufs.at[0]], buf, sem).wait()

    def scatter_start(buf, b, sem):
      pltpu.async_copy(buf, agg_sh.at[dbufs.at[b]], sem, add=True)

    def wait_scatter(buf, sem):
      pltpu.make_async_copy(buf, agg_sh.at[dbufs.at[0]], sem).wait()

    load_idx_sync(0, 0)
    gather(0, rows0, gsem0)
    for b in (1, 2, 3):
      load_idx_async(b, b)

    def step(kk, t, wait_prev=True, prefetch=True, issue_gather=True):
      rb = rbufs[t % 2]
      wait_gather(rb, gsems[t % 2])
      scatter_start(rb, t, ssems[t % 2])
      if wait_prev:
        ro = rbufs[(t + 1) % 2]
        wait_scatter(ro, ssems[(t + 1) % 2])
      if issue_gather:
        tn = (t + 1) % 8
        wait_idx(tn)
        gather(tn, rbufs[(t + 1) % 2], gsems[(t + 1) % 2])
      if prefetch:
        load_idx_async(kk + 4, (t + 4) % 8)

    for t in range(8):
      step(t, t, wait_prev=(t >= 1))

    def body(jj, _):
      kk = 8 * (jj + 1)
      for t in range(8):
        step(kk + t, t)
      return 0

    lax.fori_loop(0, (NCH - 16) // 8, body, 0)
    for t in range(8):
      kk = NCH - 8 + t
      step(kk, t, prefetch=(kk + 4 < NCH), issue_gather=(kk + 1 < NCH))
    wait_scatter(rbufs[(NCH - 1) % 2], ssems[(NCH - 1) % 2])
    plsc.subcore_barrier()

    rr = pl.ds(base_row, ROWS_PER_TILE)

    @pl.when(c == 0)
    def _():
      pltpu.sync_copy(agg_sh.at[rr], aggc0_hbm.at[rr])

    @pl.when(c == 1)
    def _():
      pltpu.sync_copy(agg_sh.at[rr], aggc1_hbm.at[rr])

  return k(src2, dst2, u2e)


def _stage_b(users, u2e, aggc0, aggc1):
  mesh = plsc.VectorSubcoreMesh(core_axis_name="c", subcore_axis_name="s")
  f32 = jnp.float32

  @functools.partial(
      pl.kernel,
      mesh=mesh,
      compiler_params=pltpu.CompilerParams(use_tc_tiling_on_sc=False),
      out_type=[
          jax.ShapeDtypeStruct((BATCH, EMBED_DIM), f32),
          jax.ShapeDtypeStruct((BATCH, EMBED_DIM), f32),
          jax.ShapeDtypeStruct((BATCH, EMBED_DIM), f32),
          jax.ShapeDtypeStruct((BATCH, 16), f32),
      ],
      scratch_types=[
          pltpu.VMEM((2, UCH), jnp.int32),
          pltpu.VMEM((UCH, EMBED_DIM), f32),
          pltpu.VMEM((UCH, EMBED_DIM), f32),
          pltpu.VMEM((UCH, DCOL), f32),
          pltpu.VMEM((UCH, DCOL), f32),
          pltpu.VMEM((UCH, DCOL), f32),
          pltpu.VMEM((UCH, DCOL), f32),
          pltpu.VMEM((UCH, 16), f32),
          pltpu.VMEM((UCH, 16), f32),
          pltpu.SemaphoreType.DMA,
          pltpu.SemaphoreType.DMA,
          pltpu.SemaphoreType.DMA,
          pltpu.SemaphoreType.DMA,
      ],
  )
  def k(users_hbm, u2e_hbm, aggc0_hbm, aggc1_hbm,
        self_hbm, n0_hbm, n1_hbm, rec_hbm,
        uidx, sb0, sb1, a00, a01, a10, a11, rc0, rc1,
        gsem0, gsem1, wsem0, wsem1):
    c = lax.axis_index("c")
    s = lax.axis_index("s")
    wid = s * NC + c
    sets = ((sb0, a00, a10, rc0, gsem0, wsem0),
            (sb1, a01, a11, rc1, gsem1, wsem1))

    def base(kk):
      return wid * UPW + kk * UCH

    def load_uidx(kk, p):
      pltpu.sync_copy(users_hbm.at[pl.ds(base(kk), UCH)], uidx.at[p])

    def start_gathers(kk, p):
      sb, a0, a1, _, gsem, _ = sets[p]
      pltpu.async_copy(u2e_hbm.at[uidx.at[p]], sb, gsem)
      pltpu.async_copy(aggc0_hbm.at[uidx.at[p]], a0, gsem)
      pltpu.async_copy(aggc1_hbm.at[uidx.at[p]], a1, gsem)

    def wait_gathers(p):
      sb, a0, a1, _, gsem, _ = sets[p]
      pltpu.make_async_copy(u2e_hbm.at[uidx.at[p]], sb, gsem).wait()
      pltpu.make_async_copy(aggc0_hbm.at[uidx.at[p]], a0, gsem).wait()
      pltpu.make_async_copy(aggc1_hbm.at[uidx.at[p]], a1, gsem).wait()

    def start_writes(kk, p):
      sb, a0, a1, rc, _, wsem = sets[p]
      bb = pl.ds(base(kk), UCH)
      pltpu.async_copy(sb, self_hbm.at[bb], wsem)
      pltpu.async_copy(a0.at[pl.ds(0, UCH), pl.ds(0, EMBED_DIM)],
                       n0_hbm.at[bb], wsem)
      pltpu.async_copy(a1.at[pl.ds(0, UCH), pl.ds(0, EMBED_DIM)],
                       n1_hbm.at[bb], wsem)
      pltpu.async_copy(rc, rec_hbm.at[bb], wsem)

    def wait_writes(p):
      sb, a0, a1, rc, _, wsem = sets[p]
      bb = pl.ds(0, UCH)
      pltpu.make_async_copy(sb, self_hbm.at[pl.ds(0, UCH)], wsem).wait()
      pltpu.make_async_copy(a0.at[pl.ds(0, UCH), pl.ds(0, EMBED_DIM)],
                            n0_hbm.at[pl.ds(0, UCH)], wsem).wait()
      pltpu.make_async_copy(a1.at[pl.ds(0, UCH), pl.ds(0, EMBED_DIM)],
                            n1_hbm.at[pl.ds(0, UCH)], wsem).wait()
      pltpu.make_async_copy(rc, rec_hbm.at[pl.ds(0, UCH)], wsem).wait()

    def recip(p):
      _, a0, a1, rc, _, _ = sets[p]

      def rbody(i, _):
        dv = (a0[i, pl.ds(EMBED_DIM, 16)] + a1[i, pl.ds(EMBED_DIM, 16)])
        rc[i, pl.ds(0, 16)] = 1.0 / jnp.maximum(dv, 1.0)
        return 0

      lax.fori_loop(0, UCH, rbody, 0)

    load_uidx(0, 0)
    start_gathers(0, 0)
    for kk in range(NUCH):
      p = kk % 2
      if kk + 1 < NUCH:
        p1 = (kk + 1) % 2
        if kk >= 1:
          wait_writes(p1)
        load_uidx(kk + 1, p1)
        start_gathers(kk + 1, p1)
      wait_gathers(p)
      recip(p)
      start_writes(kk, p)
    wait_writes(0)
    wait_writes(1)

  return k(users, u2e, aggc0, aggc1)


def _bdot(a, b):
  return jnp.dot(a, b, preferred_element_type=jnp.float32)


def _stage_c_body(self_ref, n0_ref, n1_ref, rec_ref,
                  w_enc_ref, b_enc_ref, attn_w1_ref, attn_b1_ref,
                  attn_w2_ref, fc1_w_ref, fc1_b_ref, fc2_w_ref, fc2_b_ref,
                  gamma_ref, beta_ref, out_ref):
  neigh = (n0_ref[...] + n1_ref[...]) * rec_ref[:, 0:1]
  w_enc = w_enc_ref[...]
  h = (_bdot(self_ref[...], w_enc[:EMBED_DIM])
       + _bdot(neigh, w_enc[EMBED_DIM:])
       + b_enc_ref[...])
  h = jnp.maximum(h, 0.0)
  t = jnp.tanh(_bdot(h, attn_w1_ref[...]) + attn_b1_ref[...])
  e = _bdot(t, attn_w2_ref[...])
  m = jnp.max(e)
  a = jnp.exp(e - m)
  alpha = a / jnp.sum(a)
  x = _bdot(h * alpha, fc1_w_ref[...])
  x = x + fc1_b_ref[...]
  mu = jnp.mean(x, axis=0, keepdims=True)
  xc = x - mu
  var = jnp.mean(xc * xc, axis=0, keepdims=True)
  xn = xc * lax.rsqrt(var + 1e-5) * gamma_ref[...] + beta_ref[...]
  xr = jnp.maximum(xn, 0.0)
  out_ref[...] = _bdot(xr, fc2_w_ref[...]) + fc2_b_ref[...]


def _stage_c(self_feat, n0, n1, rec, w_enc, b_enc, attn_w1, attn_b1,
             attn_w2, fc1_w, fc1_b, fc2_w, fc2_b, gamma, beta):
  f32 = jnp.float32
  return pl.pallas_call(
      _stage_c_body,
      out_shape=jax.ShapeDtypeStruct((BATCH, EMBED_DIM), f32),
      compiler_params=pltpu.CompilerParams(
          vmem_limit_bytes=100 * 1024 * 1024),
  )(self_feat, n0, n1, rec, w_enc, b_enc, attn_w1, attn_b1, attn_w2,
    fc1_w, fc1_b, fc2_w, fc2_b, gamma, beta)


def kernel(users, edge_index, u2e, W_enc, b_enc, attn_W1, attn_b1, attn_w2,
           fc1_W, fc1_b, fc2_W, fc2_b, bn_gamma, bn_beta):
  users = users.astype(jnp.int32)
  edge_index = edge_index.astype(jnp.int32)
  src2, dst2 = _edge_fmt(edge_index)
  aggc0, aggc1 = _stage_a(src2, dst2, u2e)
  self_feat, n0, n1, rec = _stage_b(users, u2e, aggc0, aggc1)
  return _stage_c(
      self_feat, n0, n1, rec,
      W_enc, b_enc.reshape(1, EMBED_DIM),
      attn_W1, attn_b1.reshape(1, EMBED_DIM), attn_w2.reshape(EMBED_DIM, 1),
      fc1_W, fc1_b.reshape(1, EMBED_DIM),
      fc2_W, fc2_b.reshape(1, EMBED_DIM),
      bn_gamma.reshape(1, EMBED_DIM), bn_beta.reshape(1, EMBED_DIM))


# speedup vs baseline: 1.0938x; 1.0938x over previous
import functools

import jax
import jax.numpy as jnp
from jax import lax
from jax.experimental import pallas as pl
from jax.experimental.pallas import tpu as pltpu
from jax.experimental.pallas import tpu_sc as plsc

NUM_USERS = 10000
EMBED_DIM = 128
NUM_EDGES = 320000
BATCH = 16384

NC = 2
NS = 16
NW = NC * NS

DCOL = EMBED_DIM + 16

ECH = 128
NCH = 80
EPW = NCH * ECH
NROW = NW * NCH
EPAD = NROW * ECH

UPW = BATCH // NW
UCH = 64
NUCH = UPW // UCH

UPAD = 10240
ROWS_PER_TILE = UPAD // NS
ZR = 128
SROW = NUM_USERS // NS
SCH = 125

EGRID = 32
EBLK = NROW // EGRID * ECH


def _edge_fmt_body(ei_ref, src_ref, dst_ref):
  i = pl.program_id(0)
  rows = EBLK // ECH
  s = ei_ref[0:1, :].reshape(rows, ECH)
  d = ei_ref[1:2, :].reshape(rows, ECH)
  flat = (i * EBLK
          + jax.lax.broadcasted_iota(jnp.int32, (rows, ECH), 0) * ECH
          + jax.lax.broadcasted_iota(jnp.int32, (rows, ECH), 1))
  valid = flat < NUM_EDGES
  src_ref[...] = jnp.where(valid, s, (flat * 37) % NUM_USERS)
  dst_ref[...] = jnp.where(valid, d,
                           NUM_USERS + flat % (UPAD - NUM_USERS))


def _edge_fmt(edge_index):
  i32 = jnp.int32
  return pl.pallas_call(
      _edge_fmt_body,
      grid=(EGRID,),
      in_specs=[pl.BlockSpec((2, EBLK), lambda i: (0, i))],
      out_specs=[pl.BlockSpec((EBLK // ECH, ECH), lambda i: (i, 0)),
                 pl.BlockSpec((EBLK // ECH, ECH), lambda i: (i, 0))],
      out_shape=[jax.ShapeDtypeStruct((NROW, ECH), i32),
                 jax.ShapeDtypeStruct((NROW, ECH), i32)],
  )(edge_index)


def _zero_fill(ref, rows, cols):
  z = jnp.zeros((16,), jnp.float32)

  def body(i, _):
    r = i // (cols // 16)
    cidx = (i % (cols // 16)) * 16
    ref[r, pl.ds(cidx, 16)] = z
    return 0

  lax.fori_loop(0, rows * (cols // 16), body, 0)


def _stage_a(src2, dst2, u2e):
  mesh = plsc.VectorSubcoreMesh(core_axis_name="c", subcore_axis_name="s")
  f32 = jnp.float32

  @functools.partial(
      pl.kernel,
      mesh=mesh,
      compiler_params=pltpu.CompilerParams(use_tc_tiling_on_sc=False),
      out_type=[
          jax.ShapeDtypeStruct((UPAD, DCOL), f32),
          jax.ShapeDtypeStruct((UPAD, DCOL), f32),
      ],
      scratch_types=[
          pltpu.HBM((NC, NUM_USERS, DCOL), f32),
          pltpu.VMEM_SHARED((UPAD, DCOL), f32),
          pltpu.VMEM((4, ECH), jnp.int32),
          pltpu.VMEM((4, ECH), jnp.int32),
          pltpu.VMEM((ECH, DCOL), f32),
          pltpu.VMEM((ECH, DCOL), f32),
          pltpu.SemaphoreType.DMA,
          pltpu.SemaphoreType.DMA,
          pltpu.SemaphoreType.DMA,
          pltpu.SemaphoreType.DMA,
          pltpu.SemaphoreType.DMA,
          pltpu.SemaphoreType.DMA,
      ],
  )
  def k(src_hbm, dst_hbm, u2e_hbm, aggc0_hbm, aggc1_hbm,
        u2ex_hbm, agg_sh, sbufs, dbufs, rows0, rows1,
        gsem0, gsem1, isem0, isem1, isem2, isem3):
    c = lax.axis_index("c")
    s = lax.axis_index("s")
    wid = s * NC + c
    roff = wid * NCH

    one = jnp.ones((16,), f32)

    def ones_body(i, _):
      rows1[i, pl.ds(EMBED_DIM, 16)] = one
      return 0

    lax.fori_loop(0, SCH, ones_body, 0)
    for j in range(SROW // SCH):
      srow = s * SROW + j * SCH
      pltpu.sync_copy(u2e_hbm.at[pl.ds(srow, SCH)],
                      rows1.at[pl.ds(0, SCH), pl.ds(0, EMBED_DIM)])
      pltpu.sync_copy(rows1.at[pl.ds(0, SCH)],
                      u2ex_hbm.at[c, pl.ds(srow, SCH)])

    _zero_fill(rows0, ZR, DCOL)
    base_row = s * ROWS_PER_TILE
    for j in range(ROWS_PER_TILE // ZR):
      pltpu.sync_copy(rows0, agg_sh.at[pl.ds(base_row + j * ZR, ZR)])
    plsc.subcore_barrier()

    tab = u2ex_hbm.at[c]
    isems = (isem0, isem1, isem2, isem3)

    def load_idx_sync(kk, b):
      pltpu.sync_copy(src_hbm.at[roff + kk], sbufs.at[b])
      pltpu.sync_copy(dst_hbm.at[roff + kk], dbufs.at[b])

    def load_idx_async(kk, b):
      pltpu.async_copy(src_hbm.at[roff + kk], sbufs.at[b], isems[b])
      pltpu.async_copy(dst_hbm.at[roff + kk], dbufs.at[b], isems[b])

    def wait_idx(b):
      pltpu.make_async_copy(src_hbm.at[roff], sbufs.at[b], isems[b]).wait()
      pltpu.make_async_copy(dst_hbm.at[roff], dbufs.at[b], isems[b]).wait()

    def gather(b, buf, sem):
      pltpu.async_copy(tab.at[sbufs.at[b]], buf, sem)

    def wait_gather(buf, sem):
      pltpu.make_async_copy(tab.at[sbufs.at[0]], buf, sem).wait()

    def scatter(buf, b):
      pltpu.sync_copy(buf, agg_sh.at[dbufs.at[b]], add=True)

    load_idx_sync(0, 0)
    load_idx_sync(1, 1)
    gather(0, rows0, gsem0)
    gather(1, rows1, gsem1)
    load_idx_async(2, 2)
    load_idx_async(3, 3)

    def step(kk, b, rbuf, gsem):
      wait_gather(rbuf, gsem)
      scatter(rbuf, b)
      bn = (b + 2) % 4
      wait_idx(bn)
      gather(bn, rbuf, gsem)
      load_idx_async(kk + 4, b)

    def body(jj, _):
      kk = 4 * jj
      step(kk, 0, rows0, gsem0)
      step(kk + 1, 1, rows1, gsem1)
      step(kk + 2, 2, rows0, gsem0)
      step(kk + 3, 3, rows1, gsem1)
      return 0

    lax.fori_loop(0, (NCH - 4) // 4, body, 0)
    wait_gather(rows0, gsem0)
    scatter(rows0, 0)
    wait_idx(2)
    gather(2, rows0, gsem0)
    wait_gather(rows1, gsem1)
    scatter(rows1, 1)
    wait_idx(3)
    gather(3, rows1, gsem1)
    wait_gather(rows0, gsem0)
    scatter(rows0, 2)
    wait_gather(rows1, gsem1)
    scatter(rows1, 3)
    plsc.subcore_barrier()

    rr = pl.ds(base_row, ROWS_PER_TILE)

    @pl.when(c == 0)
    def _():
      pltpu.sync_copy(agg_sh.at[rr], aggc0_hbm.at[rr])

    @pl.when(c == 1)
    def _():
      pltpu.sync_copy(agg_sh.at[rr], aggc1_hbm.at[rr])

  return k(src2, dst2, u2e)


def _stage_b(users, u2e, aggc0, aggc1):
  mesh = plsc.VectorSubcoreMesh(core_axis_name="c", subcore_axis_name="s")
  f32 = jnp.float32

  @functools.partial(
      pl.kernel,
      mesh=mesh,
      compiler_params=pltpu.CompilerParams(use_tc_tiling_on_sc=False),
      out_type=[
          jax.ShapeDtypeStruct((BATCH, EMBED_DIM), f32),
          jax.ShapeDtypeStruct((BATCH, EMBED_DIM), f32),
          jax.ShapeDtypeStruct((BATCH, EMBED_DIM), f32),
          jax.ShapeDtypeStruct((BATCH, 16), f32),
      ],
      scratch_types=[
          pltpu.VMEM((2, UCH), jnp.int32),
          pltpu.VMEM((UCH, EMBED_DIM), f32),
          pltpu.VMEM((UCH, EMBED_DIM), f32),
          pltpu.VMEM((UCH, DCOL), f32),
          pltpu.VMEM((UCH, DCOL), f32),
          pltpu.VMEM((UCH, DCOL), f32),
          pltpu.VMEM((UCH, DCOL), f32),
          pltpu.VMEM((UCH, 16), f32),
          pltpu.VMEM((UCH, 16), f32),
          pltpu.SemaphoreType.DMA,
          pltpu.SemaphoreType.DMA,
          pltpu.SemaphoreType.DMA,
          pltpu.SemaphoreType.DMA,
      ],
  )
  def k(users_hbm, u2e_hbm, aggc0_hbm, aggc1_hbm,
        self_hbm, n0_hbm, n1_hbm, rec_hbm,
        uidx, sb0, sb1, a00, a01, a10, a11, rc0, rc1,
        gsem0, gsem1, wsem0, wsem1):
    c = lax.axis_index("c")
    s = lax.axis_index("s")
    wid = s * NC + c
    sets = ((sb0, a00, a10, rc0, gsem0, wsem0),
            (sb1, a01, a11, rc1, gsem1, wsem1))

    def base(kk):
      return wid * UPW + kk * UCH

    def load_uidx(kk, p):
      pltpu.sync_copy(users_hbm.at[pl.ds(base(kk), UCH)], uidx.at[p])

    def start_gathers(kk, p):
      sb, a0, a1, _, gsem, _ = sets[p]
      pltpu.async_copy(u2e_hbm.at[uidx.at[p]], sb, gsem)
      pltpu.async_copy(aggc0_hbm.at[uidx.at[p]], a0, gsem)
      pltpu.async_copy(aggc1_hbm.at[uidx.at[p]], a1, gsem)

    def wait_gathers(p):
      sb, a0, a1, _, gsem, _ = sets[p]
      pltpu.make_async_copy(u2e_hbm.at[uidx.at[p]], sb, gsem).wait()
      pltpu.make_async_copy(aggc0_hbm.at[uidx.at[p]], a0, gsem).wait()
      pltpu.make_async_copy(aggc1_hbm.at[uidx.at[p]], a1, gsem).wait()

    def start_writes(kk, p):
      sb, a0, a1, rc, _, wsem = sets[p]
      bb = pl.ds(base(kk), UCH)
      pltpu.async_copy(sb, self_hbm.at[bb], wsem)
      pltpu.async_copy(a0.at[pl.ds(0, UCH), pl.ds(0, EMBED_DIM)],
                       n0_hbm.at[bb], wsem)
      pltpu.async_copy(a1.at[pl.ds(0, UCH), pl.ds(0, EMBED_DIM)],
                       n1_hbm.at[bb], wsem)
      pltpu.async_copy(rc, rec_hbm.at[bb], wsem)

    def wait_writes(p):
      sb, a0, a1, rc, _, wsem = sets[p]
      bb = pl.ds(0, UCH)
      pltpu.make_async_copy(sb, self_hbm.at[pl.ds(0, UCH)], wsem).wait()
      pltpu.make_async_copy(a0.at[pl.ds(0, UCH), pl.ds(0, EMBED_DIM)],
                            n0_hbm.at[pl.ds(0, UCH)], wsem).wait()
      pltpu.make_async_copy(a1.at[pl.ds(0, UCH), pl.ds(0, EMBED_DIM)],
                            n1_hbm.at[pl.ds(0, UCH)], wsem).wait()
      pltpu.make_async_copy(rc, rec_hbm.at[pl.ds(0, UCH)], wsem).wait()

    def recip(p):
      _, a0, a1, rc, _, _ = sets[p]

      def rbody(i, _):
        dv = (a0[i, pl.ds(EMBED_DIM, 16)] + a1[i, pl.ds(EMBED_DIM, 16)])
        rc[i, pl.ds(0, 16)] = 1.0 / jnp.maximum(dv, 1.0)
        return 0

      lax.fori_loop(0, UCH, rbody, 0)

    load_uidx(0, 0)
    start_gathers(0, 0)
    for kk in range(NUCH):
      p = kk % 2
      if kk + 1 < NUCH:
        p1 = (kk + 1) % 2
        if kk >= 1:
          wait_writes(p1)
        load_uidx(kk + 1, p1)
        start_gathers(kk + 1, p1)
      wait_gathers(p)
      recip(p)
      start_writes(kk, p)
    wait_writes(0)
    wait_writes(1)

  return k(users, u2e, aggc0, aggc1)


def _bdot(a, b):
  return jnp.dot(a, b, preferred_element_type=jnp.float32)


def _stage_c_body(self_ref, n0_ref, n1_ref, rec_ref,
                  w_enc_ref, b_enc_ref, attn_w1_ref, attn_b1_ref,
                  attn_w2_ref, fc1_w_ref, fc1_b_ref, fc2_w_ref, fc2_b_ref,
                  gamma_ref, beta_ref, out_ref):
  neigh = (n0_ref[...] + n1_ref[...]) * rec_ref[:, 0:1]
  w_enc = w_enc_ref[...]
  h = (_bdot(self_ref[...], w_enc[:EMBED_DIM])
       + _bdot(neigh, w_enc[EMBED_DIM:])
       + b_enc_ref[...])
  h = jnp.maximum(h, 0.0)
  t = jnp.tanh(_bdot(h, attn_w1_ref[...]) + attn_b1_ref[...])
  e = _bdot(t, attn_w2_ref[...])
  m = jnp.max(e)
  a = jnp.exp(e - m)
  alpha = a / jnp.sum(a)
  x = _bdot(h * alpha, fc1_w_ref[...])
  x = x + fc1_b_ref[...]
  mu = jnp.mean(x, axis=0, keepdims=True)
  xc = x - mu
  var = jnp.mean(xc * xc, axis=0, keepdims=True)
  xn = xc * lax.rsqrt(var + 1e-5) * gamma_ref[...] + beta_ref[...]
  xr = jnp.maximum(xn, 0.0)
  out_ref[...] = _bdot(xr, fc2_w_ref[...]) + fc2_b_ref[...]


def _stage_c(self_feat, n0, n1, rec, w_enc, b_enc, attn_w1, attn_b1,
             attn_w2, fc1_w, fc1_b, fc2_w, fc2_b, gamma, beta):
  f32 = jnp.float32
  return pl.pallas_call(
      _stage_c_body,
      out_shape=jax.ShapeDtypeStruct((BATCH, EMBED_DIM), f32),
      compiler_params=pltpu.CompilerParams(
          vmem_limit_bytes=100 * 1024 * 1024),
  )(self_feat, n0, n1, rec, w_enc, b_enc, attn_w1, attn_b1, attn_w2,
    fc1_w, fc1_b, fc2_w, fc2_b, gamma, beta)


def kernel(users, edge_index, u2e, W_enc, b_enc, attn_W1, attn_b1, attn_w2,
           fc1_W, fc1_b, fc2_W, fc2_b, bn_gamma, bn_beta):
  users = users.astype(jnp.int32)
  edge_index = edge_index.astype(jnp.int32)
  src2, dst2 = _edge_fmt(edge_index)
  aggc0, aggc1 = _stage_a(src2, dst2, u2e)
  self_feat, n0, n1, rec = _stage_b(users, u2e, aggc0, aggc1)
  return _stage_c(
      self_feat, n0, n1, rec,
      W_enc, b_enc.reshape(1, EMBED_DIM),
      attn_W1, attn_b1.reshape(1, EMBED_DIM), attn_w2.reshape(EMBED_DIM, 1),
      fc1_W, fc1_b.reshape(1, EMBED_DIM),
      fc2_W, fc2_b.reshape(1, EMBED_DIM),
      bn_gamma.reshape(1, EMBED_DIM), bn_beta.reshape(1, EMBED_DIM))
